# baseline (speedup 1.0000x reference)
import jax
import jax.numpy as jnp
import numpy as np
from jax.experimental import pallas as pl
from jax.experimental.pallas import tpu as pltpu

_B = 64
_C = 3
_BI = 3
_H = 512
_PS = 16
_NP = _H // _PS
_GB = 1
_NS = _B // _GB

_rows = np.arange(_C * _NP)
_cols = np.arange(_C * _H)
_LMAT = (
    ((_cols[None, :] // _H) == (_rows[:, None] // _NP))
    & (((_cols[None, :] % _H) // _PS) == (_rows[:, None] % _NP))
).astype(np.float32)
_RMAT = ((np.arange(_H)[:, None] // _PS) == np.arange(_NP)[None, :]).astype(
    np.float32
)


def _mpp_kernel(tgt_ref, pred_ref, mask_ref, lmat_ref, rmat_ref, out_ref):
    loss = jnp.zeros((1, 1), jnp.float32)
    cnt = jnp.zeros((1, 1), jnp.float32)
    for i in range(_GB):
        t = tgt_ref[i]
        ts = jnp.sum(t.reshape(_C * _NP, _PS, _H), axis=1)
        s = jnp.dot(ts, rmat_ref[...], preferred_element_type=jnp.float32)
        avg = s * (1.0 / (_PS * _PS))

        idx = (
            (avg >= 0.333).astype(jnp.int32)
            + (avg >= 0.666).astype(jnp.int32)
            + (avg >= 1.0).astype(jnp.int32)
        )

        m = mask_ref[i]
        m3 = jnp.broadcast_to(m[None], (_C, _NP, _NP)).reshape(_C * _NP, _NP)

        for k in range(_BI):
            oh = (idx == k).astype(jnp.float32)
            d = pred_ref[i, k] - oh
            loss = loss + jnp.sum(d * d * m3, keepdims=True)
        cnt = cnt + jnp.sum(m, keepdims=True)

    out_ref[0] = jnp.concatenate([loss, cnt], axis=1)


def kernel(predicted_patches, target, mask):
    tgt = target.reshape(_B, _C * _H, _H)
    pred = predicted_patches.reshape(_B, _NP, _NP, _C, _BI)
    pred = jnp.transpose(pred, (0, 4, 3, 1, 2)).reshape(_B, _BI, _C * _NP, _NP)
    mask_f = mask.astype(jnp.float32).reshape(_B, _NP, _NP)

    out = pl.pallas_call(
        _mpp_kernel,
        grid=(_NS,),
        in_specs=[
            pl.BlockSpec((_GB, _C * _H, _H), lambda b: (b, 0, 0)),
            pl.BlockSpec((_GB, _BI, _C * _NP, _NP), lambda b: (b, 0, 0, 0)),
            pl.BlockSpec((_GB, _NP, _NP), lambda b: (b, 0, 0)),
            pl.BlockSpec((_C * _NP, _C * _H), lambda b: (0, 0)),
            pl.BlockSpec((_H, _NP), lambda b: (0, 0)),
        ],
        out_specs=pl.BlockSpec((1, 1, 2), lambda b: (b, 0, 0)),
        out_shape=jax.ShapeDtypeStruct((_NS, 1, 2), jnp.float32),
        compiler_params=pltpu.CompilerParams(
            dimension_semantics=("parallel",),
        ),
    )(tgt, pred, mask_f, jnp.asarray(_LMAT), jnp.asarray(_RMAT))
    num = jnp.sum(out[:, 0, 0])
    den = jnp.maximum(jnp.sum(out[:, 0, 1]) * (_C * _BI), 1.0)
    return num / den


# speedup vs baseline: 1.3089x; 1.3089x over previous
import jax
import jax.numpy as jnp
import numpy as np
from jax.experimental import pallas as pl
from jax.experimental.pallas import tpu as pltpu

_B = 64
_C = 3
_BI = 3
_H = 512
_PS = 16
_NP = _H // _PS
_GB = 4
_NS = _B // _GB

_RMAT = ((np.arange(_H)[:, None] // _PS) == np.arange(_NP)[None, :]).astype(
    np.float32
)


def _mpp_kernel(tgt_ref, pred_ref, mask_ref, rmat_ref, out_ref):
    loss = jnp.zeros((1, 1), jnp.float32)
    cnt = jnp.zeros((1, 1), jnp.float32)
    for i in range(_GB):
        t = tgt_ref[i]
        ts = jnp.sum(t.reshape(_C * _NP, _PS, _H), axis=1)
        s = jnp.dot(ts, rmat_ref[...], preferred_element_type=jnp.float32)
        avg = s * (1.0 / (_PS * _PS))

        idx = (
            (avg >= 0.333).astype(jnp.int32)
            + (avg >= 0.666).astype(jnp.int32)
            + (avg >= 1.0).astype(jnp.int32)
        )

        m = mask_ref[i]
        m3 = jnp.broadcast_to(m[None], (_C, _NP, _NP)).reshape(_C * _NP, _NP)

        for k in range(_BI):
            oh = (idx == k).astype(jnp.float32)
            d = pred_ref[i, k] - oh
            loss = loss + jnp.sum(d * d * m3, keepdims=True)
        cnt = cnt + jnp.sum(m, keepdims=True)

    out_ref[0] = jnp.concatenate([loss, cnt], axis=1)


def kernel(predicted_patches, target, mask):
    tgt = target.reshape(_B, _C * _H, _H)
    pred = predicted_patches.reshape(_B, _NP, _NP, _C, _BI)
    pred = jnp.transpose(pred, (0, 4, 3, 1, 2)).reshape(_B, _BI, _C * _NP, _NP)
    mask_f = mask.astype(jnp.float32).reshape(_B, _NP, _NP)

    out = pl.pallas_call(
        _mpp_kernel,
        grid=(_NS,),
        in_specs=[
            pl.BlockSpec((_GB, _C * _H, _H), lambda b: (b, 0, 0)),
            pl.BlockSpec((_GB, _BI, _C * _NP, _NP), lambda b: (b, 0, 0, 0)),
            pl.BlockSpec((_GB, _NP, _NP), lambda b: (b, 0, 0)),
            pl.BlockSpec((_H, _NP), lambda b: (0, 0)),
        ],
        out_specs=pl.BlockSpec((1, 1, 2), lambda b: (b, 0, 0)),
        out_shape=jax.ShapeDtypeStruct((_NS, 1, 2), jnp.float32),
        compiler_params=pltpu.CompilerParams(
            dimension_semantics=("parallel",),
        ),
    )(tgt, pred, mask_f, jnp.asarray(_RMAT))
    num = jnp.sum(out[:, 0, 0])
    den = jnp.maximum(jnp.sum(out[:, 0, 1]) * (_C * _BI), 1.0)
    return num / den
